# baseline (speedup 1.0000x reference)
import functools

import jax
import jax.numpy as jnp
from jax import lax
from jax.experimental import pallas as pl
from jax.experimental.pallas import tpu as pltpu


def _make_tc_gather(V, D, N, R):
    assert N % R == 0 and D % 128 == 0
    G = N // R
    SL = 8
    LN = D // SL

    def body(ids_ref, *refs):
        out_ref = refs[R]
        for j in range(R):
            out_ref[j] = refs[j][0]

    grid_spec = pltpu.PrefetchScalarGridSpec(
        num_scalar_prefetch=1,
        grid=(G,),
        in_specs=[
            pl.BlockSpec(
                (1, SL, LN),
                functools.partial(
                    lambda i, ids, j=0: (ids[i * R + j], 0, 0), j=j)
            )
            for j in range(R)
        ],
        out_specs=pl.BlockSpec((R, SL, LN), lambda i, ids: (i, 0, 0)),
    )
    return pl.pallas_call(
        body,
        grid_spec=grid_spec,
        out_shape=jax.ShapeDtypeStruct((N, SL, LN), jnp.float32),
    )


def kernel(input_ids, attention_mask, embed_table):
    del attention_mask
    V, D = embed_table.shape
    B_, S = input_ids.shape
    N = B_ * S
    R = 16
    ids = input_ids.reshape(N)
    fn = _make_tc_gather(V, D, N, R)
    tbl = embed_table.reshape(V, 8, D // 8)
    out = fn(ids, *([tbl] * R))
    return out.reshape(B_, S, D)


# speedup vs baseline: 14.7143x; 14.7143x over previous
import functools

import jax
import jax.numpy as jnp
from jax import lax
from jax.experimental import pallas as pl
from jax.experimental.pallas import tpu as pltpu
from jax.experimental.pallas import tpu_sc as plsc

_INFO = plsc.get_sparse_core_info()
_NC, _NS = _INFO.num_cores, _INFO.num_subcores
_NW = _NC * _NS


def _make_gather(V, D, BATCH, SEQ, chunk):
    B = BATCH * SEQ
    assert B % _NW == 0
    b_per_w = B // _NW
    assert SEQ % b_per_w == 0
    w_per_row = SEQ // b_per_w
    assert b_per_w % chunk == 0
    n_chunks = b_per_w // chunk
    assert n_chunks % 2 == 0
    mesh = plsc.VectorSubcoreMesh(core_axis_name="c", subcore_axis_name="s")

    @functools.partial(
        pl.kernel,
        mesh=mesh,
        out_type=jax.ShapeDtypeStruct((B, D), jnp.float32),
        scratch_types=[
            pltpu.VMEM((b_per_w,), jnp.int32),
            pltpu.VMEM((chunk, D), jnp.float32),
            pltpu.VMEM((chunk, D), jnp.float32),
            pltpu.SemaphoreType.DMA,
            pltpu.SemaphoreType.DMA,
            pltpu.SemaphoreType.DMA,
            pltpu.SemaphoreType.DMA,
            pltpu.SemaphoreType.DMA,
        ],
    )
    def gather_kernel(table_hbm, ids_hbm, out_hbm, idx_v, buf0, buf1,
                      isem, gsem0, gsem1, wsem0, wsem1):
        wid = lax.axis_index("s") * _NC + lax.axis_index("c")
        base = wid * b_per_w
        row = wid // w_per_row
        col = (wid % w_per_row) * b_per_w
        idx_copy = pltpu.make_async_copy(
            ids_hbm.at[row, pl.ds(col, b_per_w)], idx_v, isem)
        idx_copy.start()
        idx_copy.wait()

        def gather(g, buf, gsem):
            pltpu.async_copy(
                table_hbm.at[idx_v.at[pl.ds(g * chunk, chunk)]], buf, gsem)

        def wait_gather(g, buf, gsem):
            pltpu.make_async_copy(
                table_hbm.at[idx_v.at[pl.ds(g * chunk, chunk)]], buf,
                gsem).wait()

        def write(g, buf, wsem):
            pltpu.async_copy(buf, out_hbm.at[pl.ds(base + g * chunk, chunk)],
                             wsem)

        def wait_write(g, buf, wsem):
            pltpu.make_async_copy(
                buf, out_hbm.at[pl.ds(base + g * chunk, chunk)], wsem).wait()

        gather(0, buf0, gsem0)
        gather(1, buf1, gsem1)

        def body(h, carry):
            g = h * 2
            wait_gather(g, buf0, gsem0)
            write(g, buf0, wsem0)
            wait_gather(g + 1, buf1, gsem1)
            write(g + 1, buf1, wsem1)

            @pl.when(g + 2 < n_chunks)
            def _refill():
                wait_write(g, buf0, wsem0)
                gather(g + 2, buf0, gsem0)
                wait_write(g + 1, buf1, wsem1)
                gather(g + 3, buf1, gsem1)

            return carry

        lax.fori_loop(0, n_chunks // 2, body, 0)
        wait_write(n_chunks - 2, buf0, wsem0)
        wait_write(n_chunks - 1, buf1, wsem1)

    return gather_kernel


def kernel(input_ids, attention_mask, embed_table):
    del attention_mask
    V, D = embed_table.shape
    BATCH, SEQ = input_ids.shape
    out = _make_gather(V, D, BATCH, SEQ, 32)(embed_table, input_ids)
    return out.reshape(BATCH, SEQ, D)


# speedup vs baseline: 15.0030x; 1.0196x over previous
import functools

import jax
import jax.numpy as jnp
from jax import lax
from jax.experimental import pallas as pl
from jax.experimental.pallas import tpu as pltpu
from jax.experimental.pallas import tpu_sc as plsc

_INFO = plsc.get_sparse_core_info()
_NC, _NS = _INFO.num_cores, _INFO.num_subcores
_NW = _NC * _NS


def _make_gather(V, D, BATCH, SEQ, chunk):
    B = BATCH * SEQ
    assert B % _NW == 0
    b_per_w = B // _NW
    assert SEQ % b_per_w == 0
    w_per_row = SEQ // b_per_w
    assert b_per_w % chunk == 0
    n_chunks = b_per_w // chunk
    nbuf = 4
    assert n_chunks % nbuf == 0
    mesh = plsc.VectorSubcoreMesh(core_axis_name="c", subcore_axis_name="s")

    @functools.partial(
        pl.kernel,
        mesh=mesh,
        out_type=jax.ShapeDtypeStruct((B, D), jnp.float32),
        scratch_types=[
            pltpu.VMEM((b_per_w,), jnp.int32),
        ] + [pltpu.VMEM((chunk, D), jnp.float32)] * nbuf + [
            pltpu.SemaphoreType.DMA,
        ] + [pltpu.SemaphoreType.DMA] * (2 * nbuf),
    )
    def gather_kernel(table_hbm, ids_hbm, out_hbm, idx_v, *rest):
        bufs = rest[:nbuf]
        isem = rest[nbuf]
        gsems = rest[nbuf + 1:2 * nbuf + 1]
        wsems = rest[2 * nbuf + 1:]
        wid = lax.axis_index("s") * _NC + lax.axis_index("c")
        base = wid * b_per_w
        row = wid // w_per_row
        col = (wid % w_per_row) * b_per_w
        idx_copy = pltpu.make_async_copy(
            ids_hbm.at[row, pl.ds(col, b_per_w)], idx_v, isem)
        idx_copy.start()
        idx_copy.wait()

        def gather(g, b):
            pltpu.async_copy(
                table_hbm.at[idx_v.at[pl.ds(g * chunk, chunk)]], bufs[b],
                gsems[b])

        def wait_gather(g, b):
            pltpu.make_async_copy(
                table_hbm.at[idx_v.at[pl.ds(g * chunk, chunk)]], bufs[b],
                gsems[b]).wait()

        def write(g, b):
            pltpu.async_copy(
                bufs[b], out_hbm.at[pl.ds(base + g * chunk, chunk)], wsems[b])

        def wait_write(g, b):
            pltpu.make_async_copy(
                bufs[b], out_hbm.at[pl.ds(base + g * chunk, chunk)],
                wsems[b]).wait()

        for b in range(nbuf):
            gather(b, b)

        def body(h, carry):
            g = h * nbuf
            for b in range(nbuf):
                wait_gather(g + b, b)
                write(g + b, b)

            @pl.when(g + nbuf < n_chunks)
            def _refill():
                for b in range(nbuf):
                    wait_write(g + b, b)
                    gather(g + nbuf + b, b)

            return carry

        lax.fori_loop(0, n_chunks // nbuf, body, 0)
        for b in range(nbuf):
            wait_write(n_chunks - nbuf + b, b)

    return gather_kernel


def kernel(input_ids, attention_mask, embed_table):
    del attention_mask
    V, D = embed_table.shape
    BATCH, SEQ = input_ids.shape
    out = _make_gather(V, D, BATCH, SEQ, 16)(embed_table, input_ids)
    return out.reshape(BATCH, SEQ, D)
